# baseline (speedup 1.0000x reference)
import functools

import jax
import jax.numpy as jnp
import numpy as np
from jax import lax
from jax.experimental import pallas as pl
from jax.experimental.pallas import tpu as pltpu
from jax.experimental.pallas import tpu_sc as plsc

_TAU = 0.5
_PD1, _PM1, _PD2, _PM2 = 0.2, 0.3, 0.4, 0.4

_NW = 32
_LANES = 16
_F = 128
_E = 320000


def _threefry2x32(k0, k1, x0, x1):
    x0 = x0.astype(np.uint32).copy()
    x1 = x1.astype(np.uint32).copy()
    ks0 = np.uint32(k0)
    ks1 = np.uint32(k1)
    ks2 = np.uint32(np.uint32(0x1BD11BDA) ^ ks0 ^ ks1)
    rot_a = (13, 15, 26, 6)
    rot_b = (17, 29, 16, 24)

    def rounds(x0, x1, rots):
        for r in rots:
            x0 = (x0 + x1).astype(np.uint32)
            x1 = ((x1 << np.uint32(r)) | (x1 >> np.uint32(32 - r))).astype(
                np.uint32)
            x1 = x1 ^ x0
        return x0, x1

    x0 = (x0 + ks0).astype(np.uint32)
    x1 = (x1 + ks1).astype(np.uint32)
    sched = [(rot_a, ks1, ks2), (rot_b, ks2, ks0), (rot_a, ks0, ks1),
             (rot_b, ks1, ks2), (rot_a, ks2, ks0)]
    for i, (rots, a0, a1) in enumerate(sched):
        x0, x1 = rounds(x0, x1, rots)
        x0 = (x0 + a0).astype(np.uint32)
        x1 = (x1 + a1 + np.uint32(i + 1)).astype(np.uint32)
    return x0, x1


def _np_random_bits(keydata, size):
    counts = np.arange(size, dtype=np.uint32)
    x0, x1 = _threefry2x32(keydata[0], keydata[1],
                           np.zeros(size, np.uint32), counts)
    return x0 ^ x1


def _np_split(keydata, num):
    x0, x1 = _threefry2x32(keydata[0], keydata[1],
                           np.zeros(num, np.uint32),
                           np.arange(num, dtype=np.uint32))
    return np.stack([x0, x1], axis=1)


def _np_uniform(keydata, size):
    bits = _np_random_bits(keydata, size)
    f = ((bits >> np.uint32(9)) | np.uint32(0x3F800000)).view(np.float32)
    return np.maximum(0.0, f - np.float32(1.0))


def _aug_constants():
    root = np.array([0, 42], dtype=np.uint32)
    sub = _np_split(root, 4)
    keep1 = _np_uniform(sub[0], _E) >= _PD1
    m1 = _np_uniform(sub[1], _F) >= _PM1
    keep2 = _np_uniform(sub[2], _E) >= _PD2
    m2 = _np_uniform(sub[3], _F) >= _PM2
    return (keep1, m1.astype(np.float32), keep2, m2.astype(np.float32))


_KEEP1, _M1, _KEEP2, _M2 = _aug_constants()
_IDX1 = np.nonzero(_KEEP1)[0].astype(np.int32)
_IDX2 = np.nonzero(_KEEP2)[0].astype(np.int32)


def _cdiv(a, b):
    return (a + b - 1) // b


def _sc_degree(src_h, dst_h, w1_h, w2_h, npad):
    jcl = src_h.shape[1]
    nsteps = jcl // _LANES
    rpt = npad // _LANES

    mesh = plsc.VectorSubcoreMesh(core_axis_name="c", subcore_axis_name="s")

    @functools.partial(
        pl.kernel,
        out_type=jax.ShapeDtypeStruct((2, 4, npad), jnp.float32),
        mesh=mesh,
        compiler_params=pltpu.CompilerParams(needs_layout_passes=False),
        scratch_types=[
            pltpu.VMEM((jcl,), jnp.int32),
            pltpu.VMEM((jcl,), jnp.int32),
            pltpu.VMEM((jcl,), jnp.float32),
            pltpu.VMEM((jcl,), jnp.float32),
            pltpu.VMEM((npad,), jnp.float32),
            pltpu.VMEM((npad,), jnp.float32),
            pltpu.VMEM((npad,), jnp.float32),
            pltpu.VMEM((npad,), jnp.float32),
            pltpu.VMEM((4, rpt), jnp.float32),
            pltpu.VMEM((4, rpt), jnp.float32),
            pltpu.VMEM_SHARED((16, 4, npad), jnp.float32),
        ],
    )
    def k(src_hb, dst_hb, w1_hb, w2_hb, out_hb,
          src_v, dst_v, w1_v, w2_v, dg0, dg1, dg2, dg3, acc_v, buf_v, shared):
        cid = lax.axis_index("c")
        sid = lax.axis_index("s")
        wid = sid * 2 + cid

        pltpu.sync_copy(src_hb.at[wid], src_v)
        pltpu.sync_copy(dst_hb.at[wid], dst_v)
        pltpu.sync_copy(w1_hb.at[wid], w1_v)
        pltpu.sync_copy(w2_hb.at[wid], w2_v)

        zero16 = jnp.zeros((_LANES,), jnp.float32)

        def zero_body(v, _):
            sl = pl.ds(v * _LANES, _LANES)
            for ref in (dg0, dg1, dg2, dg3):
                ref[sl] = zero16
            return _
        lax.fori_loop(0, npad // _LANES, zero_body, 0)

        def edge_body(v, _):
            sl = pl.ds(v * _LANES, _LANES)
            s16 = src_v[sl]
            d16 = dst_v[sl]
            w116 = w1_v[sl]
            w216 = w2_v[sl]
            plsc.addupdate_scatter(dg0, [s16], w116)
            plsc.addupdate_scatter(dg1, [d16], w116)
            plsc.addupdate_scatter(dg2, [s16], w216)
            plsc.addupdate_scatter(dg3, [d16], w216)
            return _
        lax.fori_loop(0, nsteps, edge_body, 0)

        for q, ref in enumerate((dg0, dg1, dg2, dg3)):
            pltpu.sync_copy(ref, shared.at[sid, q])
        plsc.subcore_barrier()

        r0 = sid * rpt

        def zacc_body(v, _):
            sl = pl.ds(v * _LANES, _LANES)
            for q in range(4):
                acc_v[q, sl] = zero16
            return _
        lax.fori_loop(0, rpt // _LANES, zacc_body, 0)

        def red_body(p, _):
            pltpu.sync_copy(shared.at[p, :, pl.ds(r0, rpt)], buf_v)

            def add_body(v, __):
                sl = pl.ds(v * _LANES, _LANES)
                for q in range(4):
                    acc_v[q, sl] = acc_v[q, sl] + buf_v[q, sl]
                return __
            lax.fori_loop(0, rpt // _LANES, add_body, 0)
            return _
        lax.fori_loop(0, 16, red_body, 0)

        pltpu.sync_copy(acc_v, out_hb.at[cid, :, pl.ds(r0, rpt)])

    return k(src_h, dst_h, w1_h, w2_h)


def _sc_propagate(m_pad, src_h, dst_h, npad):
    jc = src_h.shape[1]
    ch = src_h.shape[2]
    assert jc % 2 == 0 and ch == _F
    rpt = npad // _LANES

    mesh = plsc.VectorSubcoreMesh(core_axis_name="c", subcore_axis_name="s")

    @functools.partial(
        pl.kernel,
        out_type=jax.ShapeDtypeStruct((2, npad, _F), jnp.float32),
        mesh=mesh,
        scratch_types=[
            pltpu.VMEM((jc, ch), jnp.int32),
            pltpu.VMEM((jc, ch), jnp.int32),
            pltpu.VMEM((ch, _F), jnp.float32),
            pltpu.VMEM_SHARED((npad, _F), jnp.float32),
            pltpu.SemaphoreType.DMA,
        ],
    )
    def k(m_hb, src_hb, dst_hb, out_hb,
          src_v, dst_v, rows0, acc_sh, sem0):
        cid = lax.axis_index("c")
        sid = lax.axis_index("s")
        wid = sid * 2 + cid

        pltpu.sync_copy(src_hb.at[wid], src_v)
        pltpu.sync_copy(dst_hb.at[wid], dst_v)

        zero16 = jnp.zeros((_LANES,), jnp.float32)

        def zrow_body(r, _):
            for l in range(_F // _LANES):
                rows0[r, pl.ds(l * _LANES, _LANES)] = zero16
            return _
        lax.fori_loop(0, ch, zrow_body, 0)

        r0 = sid * rpt
        for b in range(rpt // ch):
            pltpu.sync_copy(rows0, acc_sh.at[pl.ds(r0 + b * ch, ch)])

        plsc.subcore_barrier()

        def edge_body(j, _):
            pltpu.async_copy(m_hb.at[src_v.at[j]], rows0, sem0).wait()
            pltpu.sync_copy(rows0, acc_sh.at[dst_v.at[j]], add=True)
            return _
        lax.fori_loop(0, jc, edge_body, 0)

        plsc.subcore_barrier()
        pltpu.sync_copy(acc_sh.at[pl.ds(r0, rpt)],
                        out_hb.at[cid, pl.ds(r0, rpt)])

    return k(m_pad, src_h, dst_h)


_BLK = 512
_LBLK = 400


def _tc_premix(xp, w, m_col, deg_out, npad):
    grid = npad // _BLK

    def body(x_ref, w_ref, m_ref, d_ref, o_ref):
        wm = w_ref[...] * m_ref[...]
        xw = jnp.dot(x_ref[...], wm, preferred_element_type=jnp.float32)
        s = lax.rsqrt(jnp.maximum(d_ref[...], 1.0))
        o_ref[...] = xw * s

    return pl.pallas_call(
        body,
        grid=(grid,),
        in_specs=[
            pl.BlockSpec((_BLK, _F), lambda i: (i, 0)),
            pl.BlockSpec((_F, _F), lambda i: (0, 0)),
            pl.BlockSpec((_F, 1), lambda i: (0, 0)),
            pl.BlockSpec((_BLK, 1), lambda i: (i, 0)),
        ],
        out_specs=pl.BlockSpec((_BLK, _F), lambda i: (i, 0)),
        out_shape=jax.ShapeDtypeStruct((npad, _F), jnp.float32),
    )(xp, w, m_col, deg_out)


def _tc_mid(p0, p1, deg_in, deg_out, w2, b1, npad):
    grid = npad // _BLK

    def body(p0_ref, p1_ref, di_ref, do_ref, w_ref, b_ref, o_ref):
        s_in = lax.rsqrt(jnp.maximum(di_ref[...], 1.0))
        h = (p0_ref[...] + p1_ref[...]) * s_in + b_ref[...]
        h = jnp.maximum(h, 0.0)
        hw = jnp.dot(h, w_ref[...], preferred_element_type=jnp.float32)
        s_out = lax.rsqrt(jnp.maximum(do_ref[...], 1.0))
        o_ref[...] = hw * s_out

    return pl.pallas_call(
        body,
        grid=(grid,),
        in_specs=[
            pl.BlockSpec((_BLK, _F), lambda i: (i, 0)),
            pl.BlockSpec((_BLK, _F), lambda i: (i, 0)),
            pl.BlockSpec((_BLK, 1), lambda i: (i, 0)),
            pl.BlockSpec((_BLK, 1), lambda i: (i, 0)),
            pl.BlockSpec((_F, _F), lambda i: (0, 0)),
            pl.BlockSpec((1, _F), lambda i: (0, 0)),
        ],
        out_specs=pl.BlockSpec((_BLK, _F), lambda i: (i, 0)),
        out_shape=jax.ShapeDtypeStruct((npad, _F), jnp.float32),
    )(p0, p1, deg_in, deg_out, w2, b1)


def _tc_out(p0, p1, deg_in, b2, fc1_w, fc1_b, fc2_w, fc2_b, npad):
    grid = npad // _BLK

    def body(p0_ref, p1_ref, di_ref, b2_ref, f1w_ref, f1b_ref,
             f2w_ref, f2b_ref, o_ref):
        s_in = lax.rsqrt(jnp.maximum(di_ref[...], 1.0))
        h2 = (p0_ref[...] + p1_ref[...]) * s_in + b2_ref[...]
        u = jnp.dot(h2, f1w_ref[...], preferred_element_type=jnp.float32)
        u = u + f1b_ref[...]
        u = jnp.where(u > 0.0, u, jnp.exp(u) - 1.0)
        z = jnp.dot(u, f2w_ref[...], preferred_element_type=jnp.float32)
        z = z + f2b_ref[...]
        nrm = jnp.sqrt(jnp.sum(z * z, axis=1, keepdims=True))
        o_ref[...] = z / (nrm + 1e-8)

    return pl.pallas_call(
        body,
        grid=(grid,),
        in_specs=[
            pl.BlockSpec((_BLK, _F), lambda i: (i, 0)),
            pl.BlockSpec((_BLK, _F), lambda i: (i, 0)),
            pl.BlockSpec((_BLK, 1), lambda i: (i, 0)),
            pl.BlockSpec((1, _F), lambda i: (0, 0)),
            pl.BlockSpec((_F, _F), lambda i: (0, 0)),
            pl.BlockSpec((1, _F), lambda i: (0, 0)),
            pl.BlockSpec((_F, _F), lambda i: (0, 0)),
            pl.BlockSpec((1, _F), lambda i: (0, 0)),
        ],
        out_specs=pl.BlockSpec((_BLK, _F), lambda i: (i, 0)),
        out_shape=jax.ShapeDtypeStruct((npad, _F), jnp.float32),
    )(p0, p1, deg_in, b2, fc1_w, fc1_b, fc2_w, fc2_b)


def _tc_loss_sums(z1n, z2n, n):
    grid = (n // _LBLK, n // _LBLK)
    inv_tau = 1.0 / _TAU

    def body(a1_ref, a2_ref, b1_ref, b2_ref, o_ref):
        j = pl.program_id(1)

        @pl.when(j == 0)
        def _():
            o_ref[...] = jnp.zeros_like(o_ref)

        a1, a2 = a1_ref[...], a2_ref[...]
        b1, b2 = b1_ref[...], b2_ref[...]

        def esum(p, q):
            s = lax.dot_general(p, q, (((1,), (1,)), ((), ())),
                                preferred_element_type=jnp.float32)
            return jnp.sum(jnp.exp(inv_tau * s), axis=1)

        upd = jnp.concatenate(
            [esum(a1, b1)[:, None], esum(a2, b2)[:, None],
             esum(a1, b2)[:, None], esum(a2, b1)[:, None],
             jnp.zeros((_LBLK, 4), jnp.float32)], axis=1)
        o_ref[...] = o_ref[...] + upd

    return pl.pallas_call(
        body,
        grid=grid,
        in_specs=[
            pl.BlockSpec((_LBLK, _F), lambda i, j: (i, 0)),
            pl.BlockSpec((_LBLK, _F), lambda i, j: (i, 0)),
            pl.BlockSpec((_LBLK, _F), lambda i, j: (j, 0)),
            pl.BlockSpec((_LBLK, _F), lambda i, j: (j, 0)),
        ],
        out_specs=pl.BlockSpec((_LBLK, 8), lambda i, j: (i, 0)),
        out_shape=jax.ShapeDtypeStruct((n, 8), jnp.float32),
    )(z1n, z2n, z1n, z2n)


def _tc_loss_final(z1n, z2n, sums, n):
    grid = n // _LBLK
    inv_tau = 1.0 / _TAU

    def body(a1_ref, a2_ref, s_ref, o_ref):
        i = pl.program_id(0)

        @pl.when(i == 0)
        def _():
            o_ref[...] = jnp.zeros_like(o_ref)

        a1, a2 = a1_ref[...], a2_ref[...]
        s = s_ref[...]
        d11 = jnp.sum(a1 * a1, axis=1)
        d22 = jnp.sum(a2 * a2, axis=1)
        d12 = jnp.sum(a1 * a2, axis=1)
        den1 = s[:, 0] - jnp.exp(inv_tau * d11) + s[:, 2]
        den2 = s[:, 1] - jnp.exp(inv_tau * d22) + s[:, 3]
        l_tot = (jnp.log(den1) + jnp.log(den2)) - 2.0 * inv_tau * d12
        o_ref[...] = o_ref[...] + jnp.sum(l_tot).reshape(1, 1) * (0.5 / n)

    return pl.pallas_call(
        body,
        grid=(grid,),
        in_specs=[
            pl.BlockSpec((_LBLK, _F), lambda i: (i, 0)),
            pl.BlockSpec((_LBLK, _F), lambda i: (i, 0)),
            pl.BlockSpec((_LBLK, 8), lambda i: (i, 0)),
        ],
        out_specs=pl.BlockSpec((1, 1), lambda i: (0, 0)),
        out_shape=jax.ShapeDtypeStruct((1, 1), jnp.float32),
    )(z1n, z2n, sums)


def kernel(x, edge_index, W1, b1, W2, b2, fc1_w, fc1_b, fc2_w, fc2_b):
    n, d = x.shape
    e = edge_index.shape[1]
    npad = _LANES * _cdiv(n + 1, _LANES)
    npad = _BLK * _cdiv(npad, _BLK)

    assert e == _E and d == _F
    src = edge_index[0]
    dst = edge_index[1]

    akey = jax.random.key(42)
    k1, _, k3, _ = jax.random.split(akey, 4)
    keep1 = jax.random.uniform(k1, (e,)) >= _PD1
    keep2 = jax.random.uniform(k3, (e,)) >= _PD2
    w1e = keep1.astype(jnp.float32)
    w2e = keep2.astype(jnp.float32)

    jc = _cdiv(e, _NW * _F)
    ep = _NW * jc * _F
    src_p = jnp.pad(src, (0, ep - e)).reshape(_NW, jc, _F)
    dst_p = jnp.pad(dst, (0, ep - e)).reshape(_NW, jc, _F)
    w1_p = jnp.pad(w1e, (0, ep - e)).reshape(_NW, jc * _F)
    w2_p = jnp.pad(w2e, (0, ep - e)).reshape(_NW, jc * _F)

    ch = _F
    jcp = _cdiv(e, _NW * ch)
    jcp = jcp + (jcp % 2)
    epp = _NW * jcp * ch
    dstr1 = jnp.where(keep1, dst, n)
    dstr2 = jnp.where(keep2, dst, n)
    srcp_p = jnp.pad(src, (0, epp - e)).reshape(_NW, jcp, ch)
    dstr1_p = jnp.pad(dstr1, (0, epp - e),
                      constant_values=n).reshape(_NW, jcp, ch)
    dstr2_p = jnp.pad(dstr2, (0, epp - e),
                      constant_values=n).reshape(_NW, jcp, ch)
    src1_p = src2_p = srcp_p

    xp = jnp.pad(x, ((0, npad - n), (0, 0)))
    m1c = jnp.asarray(_M1).reshape(d, 1)
    m2c = jnp.asarray(_M2).reshape(d, 1)

    degp = _sc_degree(src_p.reshape(_NW, jc * _F), dst_p.reshape(_NW, jc * _F),
                      w1_p, w2_p, npad)
    degs = degp[0] + degp[1]
    do1 = degs[0].reshape(npad, 1)
    di1 = degs[1].reshape(npad, 1)
    do2 = degs[2].reshape(npad, 1)
    di2 = degs[3].reshape(npad, 1)

    def encode(m_col, do_col, di_col, srcv_p, dstv_p):
        t1 = _tc_premix(xp, W1, m_col, do_col, npad)
        a1p = _sc_propagate(t1, srcv_p, dstv_p, npad)
        t2 = _tc_mid(a1p[0], a1p[1], di_col, do_col, W2, b1.reshape(1, _F),
                     npad)
        a2p = _sc_propagate(t2, srcv_p, dstv_p, npad)
        return _tc_out(a2p[0], a2p[1], di_col, b2.reshape(1, _F),
                       fc1_w, fc1_b.reshape(1, _F),
                       fc2_w, fc2_b.reshape(1, _F), npad)

    z1n = encode(m1c, do1, di1, src1_p, dstr1_p)[:n]
    z2n = encode(m2c, do2, di2, src2_p, dstr2_p)[:n]

    sums = _tc_loss_sums(z1n, z2n, n)
    loss = _tc_loss_final(z1n, z2n, sums, n)
    return loss.reshape(())


# speedup vs baseline: 1.3958x; 1.3958x over previous
import functools

import jax
import jax.numpy as jnp
import numpy as np
from jax import lax
from jax.experimental import pallas as pl
from jax.experimental.pallas import tpu as pltpu
from jax.experimental.pallas import tpu_sc as plsc

_TAU = 0.5
_PD1, _PM1, _PD2, _PM2 = 0.2, 0.3, 0.4, 0.4

_NW = 32
_LANES = 16
_F = 128
_E = 320000


def _threefry2x32(k0, k1, x0, x1):
    x0 = x0.astype(np.uint32).copy()
    x1 = x1.astype(np.uint32).copy()
    ks0 = np.uint32(k0)
    ks1 = np.uint32(k1)
    ks2 = np.uint32(np.uint32(0x1BD11BDA) ^ ks0 ^ ks1)
    rot_a = (13, 15, 26, 6)
    rot_b = (17, 29, 16, 24)

    def rounds(x0, x1, rots):
        for r in rots:
            x0 = (x0 + x1).astype(np.uint32)
            x1 = ((x1 << np.uint32(r)) | (x1 >> np.uint32(32 - r))).astype(
                np.uint32)
            x1 = x1 ^ x0
        return x0, x1

    x0 = (x0 + ks0).astype(np.uint32)
    x1 = (x1 + ks1).astype(np.uint32)
    sched = [(rot_a, ks1, ks2), (rot_b, ks2, ks0), (rot_a, ks0, ks1),
             (rot_b, ks1, ks2), (rot_a, ks2, ks0)]
    for i, (rots, a0, a1) in enumerate(sched):
        x0, x1 = rounds(x0, x1, rots)
        x0 = (x0 + a0).astype(np.uint32)
        x1 = (x1 + a1 + np.uint32(i + 1)).astype(np.uint32)
    return x0, x1


def _np_random_bits(keydata, size):
    counts = np.arange(size, dtype=np.uint32)
    x0, x1 = _threefry2x32(keydata[0], keydata[1],
                           np.zeros(size, np.uint32), counts)
    return x0 ^ x1


def _np_split(keydata, num):
    x0, x1 = _threefry2x32(keydata[0], keydata[1],
                           np.zeros(num, np.uint32),
                           np.arange(num, dtype=np.uint32))
    return np.stack([x0, x1], axis=1)


def _np_uniform(keydata, size):
    bits = _np_random_bits(keydata, size)
    f = ((bits >> np.uint32(9)) | np.uint32(0x3F800000)).view(np.float32)
    return np.maximum(0.0, f - np.float32(1.0))


def _aug_constants():
    root = np.array([0, 42], dtype=np.uint32)
    sub = _np_split(root, 4)
    keep1 = _np_uniform(sub[0], _E) >= _PD1
    m1 = _np_uniform(sub[1], _F) >= _PM1
    keep2 = _np_uniform(sub[2], _E) >= _PD2
    m2 = _np_uniform(sub[3], _F) >= _PM2
    return (keep1, m1.astype(np.float32), keep2, m2.astype(np.float32))


_KEEP1, _M1, _KEEP2, _M2 = _aug_constants()
_IDX1 = np.nonzero(_KEEP1)[0].astype(np.int32)
_IDX2 = np.nonzero(_KEEP2)[0].astype(np.int32)


def _cdiv(a, b):
    return (a + b - 1) // b


def _sc_degree(src_h, dst_h, w1_h, w2_h, npad):
    jcl = src_h.shape[1]
    nsteps = jcl // _LANES
    rpt = npad // _LANES

    mesh = plsc.VectorSubcoreMesh(core_axis_name="c", subcore_axis_name="s")

    @functools.partial(
        pl.kernel,
        out_type=jax.ShapeDtypeStruct((2, 4, npad), jnp.float32),
        mesh=mesh,
        compiler_params=pltpu.CompilerParams(needs_layout_passes=False),
        scratch_types=[
            pltpu.VMEM((jcl,), jnp.int32),
            pltpu.VMEM((jcl,), jnp.int32),
            pltpu.VMEM((jcl,), jnp.float32),
            pltpu.VMEM((jcl,), jnp.float32),
            pltpu.VMEM((npad,), jnp.float32),
            pltpu.VMEM((npad,), jnp.float32),
            pltpu.VMEM((npad,), jnp.float32),
            pltpu.VMEM((npad,), jnp.float32),
            pltpu.VMEM((4, rpt), jnp.float32),
            pltpu.VMEM((4, rpt), jnp.float32),
            pltpu.VMEM_SHARED((16, 4, npad), jnp.float32),
        ],
    )
    def k(src_hb, dst_hb, w1_hb, w2_hb, out_hb,
          src_v, dst_v, w1_v, w2_v, dg0, dg1, dg2, dg3, acc_v, buf_v, shared):
        cid = lax.axis_index("c")
        sid = lax.axis_index("s")
        wid = sid * 2 + cid

        pltpu.sync_copy(src_hb.at[wid], src_v)
        pltpu.sync_copy(dst_hb.at[wid], dst_v)
        pltpu.sync_copy(w1_hb.at[wid], w1_v)
        pltpu.sync_copy(w2_hb.at[wid], w2_v)

        zero16 = jnp.zeros((_LANES,), jnp.float32)

        def zero_body(v, _):
            sl = pl.ds(v * _LANES, _LANES)
            for ref in (dg0, dg1, dg2, dg3):
                ref[sl] = zero16
            return _
        lax.fori_loop(0, npad // _LANES, zero_body, 0)

        def edge_body(v, _):
            sl = pl.ds(v * _LANES, _LANES)
            s16 = src_v[sl]
            d16 = dst_v[sl]
            w116 = w1_v[sl]
            w216 = w2_v[sl]
            plsc.addupdate_scatter(dg0, [s16], w116)
            plsc.addupdate_scatter(dg1, [d16], w116)
            plsc.addupdate_scatter(dg2, [s16], w216)
            plsc.addupdate_scatter(dg3, [d16], w216)
            return _
        lax.fori_loop(0, nsteps, edge_body, 0)

        for q, ref in enumerate((dg0, dg1, dg2, dg3)):
            pltpu.sync_copy(ref, shared.at[sid, q])
        plsc.subcore_barrier()

        r0 = sid * rpt

        def zacc_body(v, _):
            sl = pl.ds(v * _LANES, _LANES)
            for q in range(4):
                acc_v[q, sl] = zero16
            return _
        lax.fori_loop(0, rpt // _LANES, zacc_body, 0)

        def red_body(p, _):
            pltpu.sync_copy(shared.at[p, :, pl.ds(r0, rpt)], buf_v)

            def add_body(v, __):
                sl = pl.ds(v * _LANES, _LANES)
                for q in range(4):
                    acc_v[q, sl] = acc_v[q, sl] + buf_v[q, sl]
                return __
            lax.fori_loop(0, rpt // _LANES, add_body, 0)
            return _
        lax.fori_loop(0, 16, red_body, 0)

        pltpu.sync_copy(acc_v, out_hb.at[cid, :, pl.ds(r0, rpt)])

    return k(src_h, dst_h, w1_h, w2_h)


def _sc_propagate(m_pad, src_h, dst_h, npad):
    jc = src_h.shape[1]
    ch = src_h.shape[2]
    assert ch == _F
    rpt = npad // _LANES

    mesh = plsc.VectorSubcoreMesh(core_axis_name="c", subcore_axis_name="s")

    @functools.partial(
        pl.kernel,
        out_type=jax.ShapeDtypeStruct((2, npad, _F), jnp.float32),
        mesh=mesh,
        scratch_types=[
            pltpu.VMEM((jc, ch), jnp.int32),
            pltpu.VMEM((jc, ch), jnp.int32),
            pltpu.VMEM((ch, _F), jnp.float32),
            pltpu.VMEM_SHARED((npad, _F), jnp.float32),
            pltpu.SemaphoreType.DMA,
        ],
    )
    def k(m_hb, src_hb, dst_hb, out_hb,
          src_v, dst_v, rows0, acc_sh, sem0):
        cid = lax.axis_index("c")
        sid = lax.axis_index("s")
        wid = sid * 2 + cid

        pltpu.sync_copy(src_hb.at[wid], src_v)
        pltpu.sync_copy(dst_hb.at[wid], dst_v)

        zero16 = jnp.zeros((_LANES,), jnp.float32)

        def zrow_body(r, _):
            for l in range(_F // _LANES):
                rows0[r, pl.ds(l * _LANES, _LANES)] = zero16
            return _
        lax.fori_loop(0, ch, zrow_body, 0)

        r0 = sid * rpt
        for b in range(rpt // ch):
            pltpu.sync_copy(rows0, acc_sh.at[pl.ds(r0 + b * ch, ch)])

        plsc.subcore_barrier()

        def edge_body(j, _):
            pltpu.async_copy(m_hb.at[src_v.at[j]], rows0, sem0).wait()
            pltpu.sync_copy(rows0, acc_sh.at[dst_v.at[j]], add=True)
            return _
        lax.fori_loop(0, jc, edge_body, 0)

        plsc.subcore_barrier()
        pltpu.sync_copy(acc_sh.at[pl.ds(r0, rpt)],
                        out_hb.at[cid, pl.ds(r0, rpt)])

    return k(m_pad, src_h, dst_h)


_BLK = 512
_LBLK = 400


def _tc_premix(xp, w, m_col, deg_out, npad):
    grid = npad // _BLK

    def body(x_ref, w_ref, m_ref, d_ref, o_ref):
        wm = w_ref[...] * m_ref[...]
        xw = jnp.dot(x_ref[...], wm, preferred_element_type=jnp.float32)
        s = lax.rsqrt(jnp.maximum(d_ref[...], 1.0))
        o_ref[...] = xw * s

    return pl.pallas_call(
        body,
        grid=(grid,),
        in_specs=[
            pl.BlockSpec((_BLK, _F), lambda i: (i, 0)),
            pl.BlockSpec((_F, _F), lambda i: (0, 0)),
            pl.BlockSpec((_F, 1), lambda i: (0, 0)),
            pl.BlockSpec((_BLK, 1), lambda i: (i, 0)),
        ],
        out_specs=pl.BlockSpec((_BLK, _F), lambda i: (i, 0)),
        out_shape=jax.ShapeDtypeStruct((npad, _F), jnp.float32),
    )(xp, w, m_col, deg_out)


def _tc_mid(p0, p1, deg_in, deg_out, w2, b1, npad):
    grid = npad // _BLK

    def body(p0_ref, p1_ref, di_ref, do_ref, w_ref, b_ref, o_ref):
        s_in = lax.rsqrt(jnp.maximum(di_ref[...], 1.0))
        h = (p0_ref[...] + p1_ref[...]) * s_in + b_ref[...]
        h = jnp.maximum(h, 0.0)
        hw = jnp.dot(h, w_ref[...], preferred_element_type=jnp.float32)
        s_out = lax.rsqrt(jnp.maximum(do_ref[...], 1.0))
        o_ref[...] = hw * s_out

    return pl.pallas_call(
        body,
        grid=(grid,),
        in_specs=[
            pl.BlockSpec((_BLK, _F), lambda i: (i, 0)),
            pl.BlockSpec((_BLK, _F), lambda i: (i, 0)),
            pl.BlockSpec((_BLK, 1), lambda i: (i, 0)),
            pl.BlockSpec((_BLK, 1), lambda i: (i, 0)),
            pl.BlockSpec((_F, _F), lambda i: (0, 0)),
            pl.BlockSpec((1, _F), lambda i: (0, 0)),
        ],
        out_specs=pl.BlockSpec((_BLK, _F), lambda i: (i, 0)),
        out_shape=jax.ShapeDtypeStruct((npad, _F), jnp.float32),
    )(p0, p1, deg_in, deg_out, w2, b1)


def _tc_out(p0, p1, deg_in, b2, fc1_w, fc1_b, fc2_w, fc2_b, npad):
    grid = npad // _BLK

    def body(p0_ref, p1_ref, di_ref, b2_ref, f1w_ref, f1b_ref,
             f2w_ref, f2b_ref, o_ref):
        s_in = lax.rsqrt(jnp.maximum(di_ref[...], 1.0))
        h2 = (p0_ref[...] + p1_ref[...]) * s_in + b2_ref[...]
        u = jnp.dot(h2, f1w_ref[...], preferred_element_type=jnp.float32)
        u = u + f1b_ref[...]
        u = jnp.where(u > 0.0, u, jnp.exp(u) - 1.0)
        z = jnp.dot(u, f2w_ref[...], preferred_element_type=jnp.float32)
        z = z + f2b_ref[...]
        nrm = jnp.sqrt(jnp.sum(z * z, axis=1, keepdims=True))
        o_ref[...] = z / (nrm + 1e-8)

    return pl.pallas_call(
        body,
        grid=(grid,),
        in_specs=[
            pl.BlockSpec((_BLK, _F), lambda i: (i, 0)),
            pl.BlockSpec((_BLK, _F), lambda i: (i, 0)),
            pl.BlockSpec((_BLK, 1), lambda i: (i, 0)),
            pl.BlockSpec((1, _F), lambda i: (0, 0)),
            pl.BlockSpec((_F, _F), lambda i: (0, 0)),
            pl.BlockSpec((1, _F), lambda i: (0, 0)),
            pl.BlockSpec((_F, _F), lambda i: (0, 0)),
            pl.BlockSpec((1, _F), lambda i: (0, 0)),
        ],
        out_specs=pl.BlockSpec((_BLK, _F), lambda i: (i, 0)),
        out_shape=jax.ShapeDtypeStruct((npad, _F), jnp.float32),
    )(p0, p1, deg_in, b2, fc1_w, fc1_b, fc2_w, fc2_b)


def _tc_loss_sums(z1n, z2n, n):
    grid = (n // _LBLK, n // _LBLK)
    inv_tau = 1.0 / _TAU

    def body(a1_ref, a2_ref, b1_ref, b2_ref, o_ref):
        j = pl.program_id(1)

        @pl.when(j == 0)
        def _():
            o_ref[...] = jnp.zeros_like(o_ref)

        a1, a2 = a1_ref[...], a2_ref[...]
        b1, b2 = b1_ref[...], b2_ref[...]

        def esum(p, q):
            s = lax.dot_general(p, q, (((1,), (1,)), ((), ())),
                                preferred_element_type=jnp.float32)
            return jnp.sum(jnp.exp(inv_tau * s), axis=1)

        upd = jnp.concatenate(
            [esum(a1, b1)[:, None], esum(a2, b2)[:, None],
             esum(a1, b2)[:, None], esum(a2, b1)[:, None],
             jnp.zeros((_LBLK, 4), jnp.float32)], axis=1)
        o_ref[...] = o_ref[...] + upd

    return pl.pallas_call(
        body,
        grid=grid,
        in_specs=[
            pl.BlockSpec((_LBLK, _F), lambda i, j: (i, 0)),
            pl.BlockSpec((_LBLK, _F), lambda i, j: (i, 0)),
            pl.BlockSpec((_LBLK, _F), lambda i, j: (j, 0)),
            pl.BlockSpec((_LBLK, _F), lambda i, j: (j, 0)),
        ],
        out_specs=pl.BlockSpec((_LBLK, 8), lambda i, j: (i, 0)),
        out_shape=jax.ShapeDtypeStruct((n, 8), jnp.float32),
    )(z1n, z2n, z1n, z2n)


def _tc_loss_final(z1n, z2n, sums, n):
    grid = n // _LBLK
    inv_tau = 1.0 / _TAU

    def body(a1_ref, a2_ref, s_ref, o_ref):
        i = pl.program_id(0)

        @pl.when(i == 0)
        def _():
            o_ref[...] = jnp.zeros_like(o_ref)

        a1, a2 = a1_ref[...], a2_ref[...]
        s = s_ref[...]
        d11 = jnp.sum(a1 * a1, axis=1)
        d22 = jnp.sum(a2 * a2, axis=1)
        d12 = jnp.sum(a1 * a2, axis=1)
        den1 = s[:, 0] - jnp.exp(inv_tau * d11) + s[:, 2]
        den2 = s[:, 1] - jnp.exp(inv_tau * d22) + s[:, 3]
        l_tot = (jnp.log(den1) + jnp.log(den2)) - 2.0 * inv_tau * d12
        o_ref[...] = o_ref[...] + jnp.sum(l_tot).reshape(1, 1) * (0.5 / n)

    return pl.pallas_call(
        body,
        grid=(grid,),
        in_specs=[
            pl.BlockSpec((_LBLK, _F), lambda i: (i, 0)),
            pl.BlockSpec((_LBLK, _F), lambda i: (i, 0)),
            pl.BlockSpec((_LBLK, 8), lambda i: (i, 0)),
        ],
        out_specs=pl.BlockSpec((1, 1), lambda i: (0, 0)),
        out_shape=jax.ShapeDtypeStruct((1, 1), jnp.float32),
    )(z1n, z2n, sums)


def kernel(x, edge_index, W1, b1, W2, b2, fc1_w, fc1_b, fc2_w, fc2_b):
    n, d = x.shape
    e = edge_index.shape[1]
    npad = _LANES * _cdiv(n + 1, _LANES)
    npad = _BLK * _cdiv(npad, _BLK)

    assert e == _E and d == _F
    src = edge_index[0]
    dst = edge_index[1]

    akey = jax.random.key(42)
    k1, _, k3, _ = jax.random.split(akey, 4)
    keep1 = jax.random.uniform(k1, (e,)) >= _PD1
    keep2 = jax.random.uniform(k3, (e,)) >= _PD2
    w1e = keep1.astype(jnp.float32)
    w2e = keep2.astype(jnp.float32)

    jc = _cdiv(e, _NW * _F)
    ep = _NW * jc * _F
    src_p = jnp.pad(src, (0, ep - e)).reshape(_NW, jc, _F)
    dst_p = jnp.pad(dst, (0, ep - e)).reshape(_NW, jc, _F)
    w1_p = jnp.pad(w1e, (0, ep - e)).reshape(_NW, jc * _F)
    w2_p = jnp.pad(w2e, (0, ep - e)).reshape(_NW, jc * _F)

    ch = _F
    jcp = _cdiv(e, _NW * ch)
    epp = _NW * jcp * ch
    dstr1 = jnp.where(keep1, dst, n)
    dstr2 = jnp.where(keep2, dst, n)
    srcp_p = jnp.pad(src, (0, epp - e)).reshape(_NW, jcp, ch)
    dstr1_p = jnp.pad(dstr1, (0, epp - e),
                      constant_values=n).reshape(_NW, jcp, ch)
    dstr2_p = jnp.pad(dstr2, (0, epp - e),
                      constant_values=n).reshape(_NW, jcp, ch)
    src1_p = src2_p = srcp_p

    xp = jnp.pad(x, ((0, npad - n), (0, 0)))
    m1c = jnp.asarray(_M1).reshape(d, 1)
    m2c = jnp.asarray(_M2).reshape(d, 1)

    degp = _sc_degree(src_p.reshape(_NW, jc * _F), dst_p.reshape(_NW, jc * _F),
                      w1_p, w2_p, npad)
    degs = degp[0] + degp[1]
    do1 = degs[0].reshape(npad, 1)
    di1 = degs[1].reshape(npad, 1)
    do2 = degs[2].reshape(npad, 1)
    di2 = degs[3].reshape(npad, 1)

    def encode(m_col, do_col, di_col, srcv_p, dstv_p):
        t1 = _tc_premix(xp, W1, m_col, do_col, npad)
        a1p = _sc_propagate(t1, srcv_p, dstv_p, npad)
        t2 = _tc_mid(a1p[0], a1p[1], di_col, do_col, W2, b1.reshape(1, _F),
                     npad)
        a2p = _sc_propagate(t2, srcv_p, dstv_p, npad)
        return _tc_out(a2p[0], a2p[1], di_col, b2.reshape(1, _F),
                       fc1_w, fc1_b.reshape(1, _F),
                       fc2_w, fc2_b.reshape(1, _F), npad)

    z1n = encode(m1c, do1, di1, src1_p, dstr1_p)[:n]
    z2n = encode(m2c, do2, di2, src2_p, dstr2_p)[:n]

    sums = _tc_loss_sums(z1n, z2n, n)
    loss = _tc_loss_final(z1n, z2n, sums, n)
    return loss.reshape(())


# speedup vs baseline: 1.4544x; 1.0419x over previous
import functools

import jax
import jax.numpy as jnp
import numpy as np
from jax import lax
from jax.experimental import pallas as pl
from jax.experimental.pallas import tpu as pltpu
from jax.experimental.pallas import tpu_sc as plsc

_TAU = 0.5
_PD1, _PM1, _PD2, _PM2 = 0.2, 0.3, 0.4, 0.4

_NW = 32
_LANES = 16
_F = 128
_E = 320000


def _threefry2x32(k0, k1, x0, x1):
    x0 = x0.astype(np.uint32).copy()
    x1 = x1.astype(np.uint32).copy()
    ks0 = np.uint32(k0)
    ks1 = np.uint32(k1)
    ks2 = np.uint32(np.uint32(0x1BD11BDA) ^ ks0 ^ ks1)
    rot_a = (13, 15, 26, 6)
    rot_b = (17, 29, 16, 24)

    def rounds(x0, x1, rots):
        for r in rots:
            x0 = (x0 + x1).astype(np.uint32)
            x1 = ((x1 << np.uint32(r)) | (x1 >> np.uint32(32 - r))).astype(
                np.uint32)
            x1 = x1 ^ x0
        return x0, x1

    x0 = (x0 + ks0).astype(np.uint32)
    x1 = (x1 + ks1).astype(np.uint32)
    sched = [(rot_a, ks1, ks2), (rot_b, ks2, ks0), (rot_a, ks0, ks1),
             (rot_b, ks1, ks2), (rot_a, ks2, ks0)]
    for i, (rots, a0, a1) in enumerate(sched):
        x0, x1 = rounds(x0, x1, rots)
        x0 = (x0 + a0).astype(np.uint32)
        x1 = (x1 + a1 + np.uint32(i + 1)).astype(np.uint32)
    return x0, x1


def _np_random_bits(keydata, size):
    counts = np.arange(size, dtype=np.uint32)
    x0, x1 = _threefry2x32(keydata[0], keydata[1],
                           np.zeros(size, np.uint32), counts)
    return x0 ^ x1


def _np_split(keydata, num):
    x0, x1 = _threefry2x32(keydata[0], keydata[1],
                           np.zeros(num, np.uint32),
                           np.arange(num, dtype=np.uint32))
    return np.stack([x0, x1], axis=1)


def _np_uniform(keydata, size):
    bits = _np_random_bits(keydata, size)
    f = ((bits >> np.uint32(9)) | np.uint32(0x3F800000)).view(np.float32)
    return np.maximum(0.0, f - np.float32(1.0))


def _aug_constants():
    root = np.array([0, 42], dtype=np.uint32)
    sub = _np_split(root, 4)
    keep1 = _np_uniform(sub[0], _E) >= _PD1
    m1 = _np_uniform(sub[1], _F) >= _PM1
    keep2 = _np_uniform(sub[2], _E) >= _PD2
    m2 = _np_uniform(sub[3], _F) >= _PM2
    return (keep1, m1.astype(np.float32), keep2, m2.astype(np.float32))


_KEEP1, _M1, _KEEP2, _M2 = _aug_constants()
_IDX1 = np.nonzero(_KEEP1)[0].astype(np.int32)
_IDX2 = np.nonzero(_KEEP2)[0].astype(np.int32)


def _cdiv(a, b):
    return (a + b - 1) // b


def _sc_degree(src_h, dst_h, w1_h, w2_h, npad):
    jcl = src_h.shape[1]
    nsteps = jcl // _LANES
    rpt = npad // _LANES

    mesh = plsc.VectorSubcoreMesh(core_axis_name="c", subcore_axis_name="s")

    @functools.partial(
        pl.kernel,
        out_type=jax.ShapeDtypeStruct((2, 4, npad), jnp.float32),
        mesh=mesh,
        compiler_params=pltpu.CompilerParams(needs_layout_passes=False),
        scratch_types=[
            pltpu.VMEM((jcl,), jnp.int32),
            pltpu.VMEM((jcl,), jnp.int32),
            pltpu.VMEM((jcl,), jnp.float32),
            pltpu.VMEM((jcl,), jnp.float32),
            pltpu.VMEM((npad,), jnp.float32),
            pltpu.VMEM((npad,), jnp.float32),
            pltpu.VMEM((npad,), jnp.float32),
            pltpu.VMEM((npad,), jnp.float32),
            pltpu.VMEM((4, rpt), jnp.float32),
            pltpu.VMEM((4, rpt), jnp.float32),
            pltpu.VMEM_SHARED((16, 4, npad), jnp.float32),
        ],
    )
    def k(src_hb, dst_hb, w1_hb, w2_hb, out_hb,
          src_v, dst_v, w1_v, w2_v, dg0, dg1, dg2, dg3, acc_v, buf_v, shared):
        cid = lax.axis_index("c")
        sid = lax.axis_index("s")
        wid = sid * 2 + cid

        pltpu.sync_copy(src_hb.at[wid], src_v)
        pltpu.sync_copy(dst_hb.at[wid], dst_v)
        pltpu.sync_copy(w1_hb.at[wid], w1_v)
        pltpu.sync_copy(w2_hb.at[wid], w2_v)

        zero16 = jnp.zeros((_LANES,), jnp.float32)

        def zero_body(v, _):
            sl = pl.ds(v * _LANES, _LANES)
            for ref in (dg0, dg1, dg2, dg3):
                ref[sl] = zero16
            return _
        lax.fori_loop(0, npad // _LANES, zero_body, 0)

        def edge_body(v, _):
            sl = pl.ds(v * _LANES, _LANES)
            s16 = src_v[sl]
            d16 = dst_v[sl]
            w116 = w1_v[sl]
            w216 = w2_v[sl]
            plsc.addupdate_scatter(dg0, [s16], w116)
            plsc.addupdate_scatter(dg1, [d16], w116)
            plsc.addupdate_scatter(dg2, [s16], w216)
            plsc.addupdate_scatter(dg3, [d16], w216)
            return _
        lax.fori_loop(0, nsteps, edge_body, 0)

        for q, ref in enumerate((dg0, dg1, dg2, dg3)):
            pltpu.sync_copy(ref, shared.at[sid, q])
        plsc.subcore_barrier()

        r0 = sid * rpt

        def zacc_body(v, _):
            sl = pl.ds(v * _LANES, _LANES)
            for q in range(4):
                acc_v[q, sl] = zero16
            return _
        lax.fori_loop(0, rpt // _LANES, zacc_body, 0)

        def red_body(p, _):
            pltpu.sync_copy(shared.at[p, :, pl.ds(r0, rpt)], buf_v)

            def add_body(v, __):
                sl = pl.ds(v * _LANES, _LANES)
                for q in range(4):
                    acc_v[q, sl] = acc_v[q, sl] + buf_v[q, sl]
                return __
            lax.fori_loop(0, rpt // _LANES, add_body, 0)
            return _
        lax.fori_loop(0, 16, red_body, 0)

        pltpu.sync_copy(acc_v, out_hb.at[cid, :, pl.ds(r0, rpt)])

    return k(src_h, dst_h, w1_h, w2_h)


def _sc_propagate(m_pad, src_h, dst_h, npad):
    jc = src_h.shape[1]
    ch = src_h.shape[2]
    assert ch == _F
    rpt = npad // _LANES

    mesh = plsc.VectorSubcoreMesh(core_axis_name="c", subcore_axis_name="s")

    @functools.partial(
        pl.kernel,
        out_type=jax.ShapeDtypeStruct((2, npad, _F), jnp.float32),
        mesh=mesh,
        scratch_types=[
            pltpu.VMEM((jc, ch), jnp.int32),
            pltpu.VMEM((jc, ch), jnp.int32),
            pltpu.VMEM((ch, _F), jnp.float32),
            pltpu.VMEM_SHARED((npad, _F), jnp.float32),
            pltpu.SemaphoreType.DMA,
        ],
    )
    def k(m_hb, src_hb, dst_hb, out_hb,
          src_v, dst_v, rows0, acc_sh, sem0):
        cid = lax.axis_index("c")
        sid = lax.axis_index("s")
        wid = sid * 2 + cid

        pltpu.sync_copy(src_hb.at[wid], src_v)
        pltpu.sync_copy(dst_hb.at[wid], dst_v)

        zero16 = jnp.zeros((_LANES,), jnp.float32)

        def zrow_body(r, _):
            for l in range(_F // _LANES):
                rows0[r, pl.ds(l * _LANES, _LANES)] = zero16
            return _
        lax.fori_loop(0, ch, zrow_body, 0)

        r0 = sid * rpt
        for b in range(rpt // ch):
            pltpu.sync_copy(rows0, acc_sh.at[pl.ds(r0 + b * ch, ch)])

        plsc.subcore_barrier()

        def edge_body(j, _):
            pltpu.async_copy(m_hb.at[src_v.at[j]], rows0, sem0).wait()
            pltpu.sync_copy(rows0, acc_sh.at[dst_v.at[j]], add=True)
            return _
        lax.fori_loop(0, jc, edge_body, 0)

        plsc.subcore_barrier()
        pltpu.sync_copy(acc_sh.at[pl.ds(r0, rpt)],
                        out_hb.at[cid, pl.ds(r0, rpt)])

    return k(m_pad, src_h, dst_h)


_BLK = 512
_LBLK = 400


def _tc_premix(xp, w, m_col, deg_out, npad):
    grid = npad // _BLK

    def body(x_ref, w_ref, m_ref, d_ref, o_ref):
        wm = w_ref[...] * m_ref[...]
        xw = jnp.dot(x_ref[...], wm, preferred_element_type=jnp.float32)
        s = lax.rsqrt(jnp.maximum(d_ref[...], 1.0))
        o_ref[...] = xw * s

    return pl.pallas_call(
        body,
        grid=(grid,),
        in_specs=[
            pl.BlockSpec((_BLK, _F), lambda i: (i, 0)),
            pl.BlockSpec((_F, _F), lambda i: (0, 0)),
            pl.BlockSpec((_F, 1), lambda i: (0, 0)),
            pl.BlockSpec((_BLK, 1), lambda i: (i, 0)),
        ],
        out_specs=pl.BlockSpec((_BLK, _F), lambda i: (i, 0)),
        out_shape=jax.ShapeDtypeStruct((npad, _F), jnp.float32),
    )(xp, w, m_col, deg_out)


def _tc_mid(p0, p1, deg_in, deg_out, w2, b1, npad):
    grid = npad // _BLK

    def body(p0_ref, p1_ref, di_ref, do_ref, w_ref, b_ref, o_ref):
        s_in = lax.rsqrt(jnp.maximum(di_ref[...], 1.0))
        h = (p0_ref[...] + p1_ref[...]) * s_in + b_ref[...]
        h = jnp.maximum(h, 0.0)
        hw = jnp.dot(h, w_ref[...], preferred_element_type=jnp.float32)
        s_out = lax.rsqrt(jnp.maximum(do_ref[...], 1.0))
        o_ref[...] = hw * s_out

    return pl.pallas_call(
        body,
        grid=(grid,),
        in_specs=[
            pl.BlockSpec((_BLK, _F), lambda i: (i, 0)),
            pl.BlockSpec((_BLK, _F), lambda i: (i, 0)),
            pl.BlockSpec((_BLK, 1), lambda i: (i, 0)),
            pl.BlockSpec((_BLK, 1), lambda i: (i, 0)),
            pl.BlockSpec((_F, _F), lambda i: (0, 0)),
            pl.BlockSpec((1, _F), lambda i: (0, 0)),
        ],
        out_specs=pl.BlockSpec((_BLK, _F), lambda i: (i, 0)),
        out_shape=jax.ShapeDtypeStruct((npad, _F), jnp.float32),
    )(p0, p1, deg_in, deg_out, w2, b1)


def _tc_out(p0, p1, deg_in, b2, fc1_w, fc1_b, fc2_w, fc2_b, npad):
    grid = npad // _BLK

    def body(p0_ref, p1_ref, di_ref, b2_ref, f1w_ref, f1b_ref,
             f2w_ref, f2b_ref, o_ref):
        s_in = lax.rsqrt(jnp.maximum(di_ref[...], 1.0))
        h2 = (p0_ref[...] + p1_ref[...]) * s_in + b2_ref[...]
        u = jnp.dot(h2, f1w_ref[...], preferred_element_type=jnp.float32)
        u = u + f1b_ref[...]
        u = jnp.where(u > 0.0, u, jnp.exp(u) - 1.0)
        z = jnp.dot(u, f2w_ref[...], preferred_element_type=jnp.float32)
        z = z + f2b_ref[...]
        nrm = jnp.sqrt(jnp.sum(z * z, axis=1, keepdims=True))
        o_ref[...] = z / (nrm + 1e-8)

    return pl.pallas_call(
        body,
        grid=(grid,),
        in_specs=[
            pl.BlockSpec((_BLK, _F), lambda i: (i, 0)),
            pl.BlockSpec((_BLK, _F), lambda i: (i, 0)),
            pl.BlockSpec((_BLK, 1), lambda i: (i, 0)),
            pl.BlockSpec((1, _F), lambda i: (0, 0)),
            pl.BlockSpec((_F, _F), lambda i: (0, 0)),
            pl.BlockSpec((1, _F), lambda i: (0, 0)),
            pl.BlockSpec((_F, _F), lambda i: (0, 0)),
            pl.BlockSpec((1, _F), lambda i: (0, 0)),
        ],
        out_specs=pl.BlockSpec((_BLK, _F), lambda i: (i, 0)),
        out_shape=jax.ShapeDtypeStruct((npad, _F), jnp.float32),
    )(p0, p1, deg_in, b2, fc1_w, fc1_b, fc2_w, fc2_b)


def _tc_loss_sums(z1n, z2n, n):
    grid = (n // _LBLK, n // _LBLK)
    inv_tau = 1.0 / _TAU

    def body(a1_ref, a2_ref, b1_ref, b2_ref, o_ref):
        j = pl.program_id(1)

        @pl.when(j == 0)
        def _():
            o_ref[...] = jnp.zeros_like(o_ref)

        a1, a2 = a1_ref[...], a2_ref[...]
        b1, b2 = b1_ref[...], b2_ref[...]

        def esum(p, q):
            s = lax.dot_general(p, q, (((1,), (1,)), ((), ())),
                                preferred_element_type=jnp.float32)
            return jnp.sum(jnp.exp(inv_tau * s), axis=1)

        upd = jnp.concatenate(
            [esum(a1, b1)[:, None], esum(a2, b2)[:, None],
             esum(a1, b2)[:, None], esum(a2, b1)[:, None],
             jnp.zeros((_LBLK, 4), jnp.float32)], axis=1)
        o_ref[...] = o_ref[...] + upd

    return pl.pallas_call(
        body,
        grid=grid,
        in_specs=[
            pl.BlockSpec((_LBLK, _F), lambda i, j: (i, 0)),
            pl.BlockSpec((_LBLK, _F), lambda i, j: (i, 0)),
            pl.BlockSpec((_LBLK, _F), lambda i, j: (j, 0)),
            pl.BlockSpec((_LBLK, _F), lambda i, j: (j, 0)),
        ],
        out_specs=pl.BlockSpec((_LBLK, 8), lambda i, j: (i, 0)),
        out_shape=jax.ShapeDtypeStruct((n, 8), jnp.float32),
    )(z1n, z2n, z1n, z2n)


def _tc_loss_final(z1n, z2n, sums, n):
    grid = n // _LBLK
    inv_tau = 1.0 / _TAU

    def body(a1_ref, a2_ref, s_ref, o_ref):
        i = pl.program_id(0)

        @pl.when(i == 0)
        def _():
            o_ref[...] = jnp.zeros_like(o_ref)

        a1, a2 = a1_ref[...], a2_ref[...]
        s = s_ref[...]
        d11 = jnp.sum(a1 * a1, axis=1)
        d22 = jnp.sum(a2 * a2, axis=1)
        d12 = jnp.sum(a1 * a2, axis=1)
        den1 = s[:, 0] - jnp.exp(inv_tau * d11) + s[:, 2]
        den2 = s[:, 1] - jnp.exp(inv_tau * d22) + s[:, 3]
        l_tot = (jnp.log(den1) + jnp.log(den2)) - 2.0 * inv_tau * d12
        o_ref[...] = o_ref[...] + jnp.sum(l_tot).reshape(1, 1) * (0.5 / n)

    return pl.pallas_call(
        body,
        grid=(grid,),
        in_specs=[
            pl.BlockSpec((_LBLK, _F), lambda i: (i, 0)),
            pl.BlockSpec((_LBLK, _F), lambda i: (i, 0)),
            pl.BlockSpec((_LBLK, 8), lambda i: (i, 0)),
        ],
        out_specs=pl.BlockSpec((1, 1), lambda i: (0, 0)),
        out_shape=jax.ShapeDtypeStruct((1, 1), jnp.float32),
    )(z1n, z2n, sums)


def kernel(x, edge_index, W1, b1, W2, b2, fc1_w, fc1_b, fc2_w, fc2_b):
    n, d = x.shape
    e = edge_index.shape[1]
    npad = _LANES * _cdiv(n + 1, _LANES)
    npad = _BLK * _cdiv(npad, _BLK)

    assert e == _E and d == _F
    src = edge_index[0]
    dst = edge_index[1]

    akey = jax.random.key(42)
    k1, _, k3, _ = jax.random.split(akey, 4)
    keep1 = jax.random.uniform(k1, (e,)) >= _PD1
    keep2 = jax.random.uniform(k3, (e,)) >= _PD2
    w1e = keep1.astype(jnp.float32)
    w2e = keep2.astype(jnp.float32)

    jc = _cdiv(e, _NW * _F)
    ep = _NW * jc * _F
    src_p = jnp.pad(src, (0, ep - e)).reshape(_NW, jc, _F)
    dst_p = jnp.pad(dst, (0, ep - e)).reshape(_NW, jc, _F)
    w1_p = jnp.pad(w1e, (0, ep - e)).reshape(_NW, jc * _F)
    w2_p = jnp.pad(w2e, (0, ep - e)).reshape(_NW, jc * _F)

    ch = _F
    jcp = _cdiv(e, _NW * ch)
    epp = _NW * jcp * ch
    tile_ids = (jnp.arange(epp, dtype=jnp.int32) // (jcp * ch)) + n
    kp1 = jnp.pad(keep1, (0, epp - e))
    kp2 = jnp.pad(keep2, (0, epp - e))
    dpad = jnp.pad(dst, (0, epp - e))
    dstr1_p = jnp.where(kp1, dpad, tile_ids).reshape(_NW, jcp, ch)
    dstr2_p = jnp.where(kp2, dpad, tile_ids).reshape(_NW, jcp, ch)
    srcp_p = jnp.pad(src, (0, epp - e)).reshape(_NW, jcp, ch)
    src1_p = src2_p = srcp_p

    xp = jnp.pad(x, ((0, npad - n), (0, 0)))
    m1c = jnp.asarray(_M1).reshape(d, 1)
    m2c = jnp.asarray(_M2).reshape(d, 1)

    degp = _sc_degree(src_p.reshape(_NW, jc * _F), dst_p.reshape(_NW, jc * _F),
                      w1_p, w2_p, npad)
    degs = degp[0] + degp[1]
    do1 = degs[0].reshape(npad, 1)
    di1 = degs[1].reshape(npad, 1)
    do2 = degs[2].reshape(npad, 1)
    di2 = degs[3].reshape(npad, 1)

    def encode(m_col, do_col, di_col, srcv_p, dstv_p):
        t1 = _tc_premix(xp, W1, m_col, do_col, npad)
        a1p = _sc_propagate(t1, srcv_p, dstv_p, npad)
        t2 = _tc_mid(a1p[0], a1p[1], di_col, do_col, W2, b1.reshape(1, _F),
                     npad)
        a2p = _sc_propagate(t2, srcv_p, dstv_p, npad)
        return _tc_out(a2p[0], a2p[1], di_col, b2.reshape(1, _F),
                       fc1_w, fc1_b.reshape(1, _F),
                       fc2_w, fc2_b.reshape(1, _F), npad)

    z1n = encode(m1c, do1, di1, src1_p, dstr1_p)[:n]
    z2n = encode(m2c, do2, di2, src2_p, dstr2_p)[:n]

    sums = _tc_loss_sums(z1n, z2n, n)
    loss = _tc_loss_final(z1n, z2n, sums, n)
    return loss.reshape(())
